# baseline (speedup 1.0000x reference)
import jax
import jax.numpy as jnp
from jax.experimental import pallas as pl
from jax.experimental.pallas import tpu as pltpu

_BM = 256


def _bmm_kernel(f_ref, a_ref, o_ref):
    o_ref[0] = jnp.dot(a_ref[0], f_ref[0], preferred_element_type=jnp.float32)


def kernel(features, A):
    B, M, K = A.shape
    N = features.shape[-1]
    return pl.pallas_call(
        _bmm_kernel,
        grid=(B, M // _BM),
        in_specs=[
            pl.BlockSpec((1, K, N), lambda b, i: (b, 0, 0)),
            pl.BlockSpec((1, _BM, K), lambda b, i: (b, i, 0)),
        ],
        out_specs=pl.BlockSpec((1, _BM, N), lambda b, i: (b, i, 0)),
        out_shape=jax.ShapeDtypeStruct((B, M, N), jnp.float32),
        compiler_params=pltpu.CompilerParams(
            dimension_semantics=("parallel", "parallel"),
        ),
    )(features, A)


# speedup vs baseline: 1.0090x; 1.0090x over previous
import jax
import jax.numpy as jnp
from jax.experimental import pallas as pl
from jax.experimental.pallas import tpu as pltpu

_BM = 256


def _bmm_kernel(f_ref, a_ref, o_ref):
    a = a_ref[0].astype(jnp.bfloat16)
    f = f_ref[0].astype(jnp.bfloat16)
    o_ref[0] = jnp.dot(a, f, preferred_element_type=jnp.float32)


def kernel(features, A):
    B, M, K = A.shape
    N = features.shape[-1]
    return pl.pallas_call(
        _bmm_kernel,
        grid=(B, M // _BM),
        in_specs=[
            pl.BlockSpec((1, K, N), lambda b, i: (b, 0, 0)),
            pl.BlockSpec((1, _BM, K), lambda b, i: (b, i, 0)),
        ],
        out_specs=pl.BlockSpec((1, _BM, N), lambda b, i: (b, i, 0)),
        out_shape=jax.ShapeDtypeStruct((B, M, N), jnp.float32),
        compiler_params=pltpu.CompilerParams(
            dimension_semantics=("parallel", "parallel"),
        ),
    )(features, A)


# speedup vs baseline: 1.4434x; 1.4305x over previous
import jax
import jax.numpy as jnp
from jax.experimental import pallas as pl
from jax.experimental.pallas import tpu as pltpu

_NS = 8
_BMS = 256


def _bmm_kernel(f_ref, *refs):
    a_refs, o_ref = refs[:_NS], refs[_NS]
    f = f_ref[0]
    for j in range(_NS):
        o_ref[0, j * _BMS:(j + 1) * _BMS, :] = jnp.dot(
            a_refs[j][0], f, preferred_element_type=jnp.float32)


def kernel(features, A):
    B, M, K = A.shape
    N = features.shape[-1]
    bm = _NS * _BMS
    a_specs = [
        pl.BlockSpec((1, _BMS, K), lambda b, i, j=j: (b, i * _NS + j, 0))
        for j in range(_NS)
    ]
    return pl.pallas_call(
        _bmm_kernel,
        grid=(B, M // bm),
        in_specs=[pl.BlockSpec((1, K, N), lambda b, i: (b, 0, 0))] + a_specs,
        out_specs=pl.BlockSpec((1, bm, N), lambda b, i: (b, i, 0)),
        out_shape=jax.ShapeDtypeStruct((B, M, N), jnp.float32),
        compiler_params=pltpu.CompilerParams(
            dimension_semantics=("parallel", "parallel"),
        ),
    )(features, *([A] * _NS))
